# baseline (speedup 1.0000x reference)
import jax
import jax.numpy as jnp
from jax.experimental import pallas as pl
from jax.experimental.pallas import tpu as pltpu

D_MODEL = 1024
HIDDEN = 4096
E = 8
B = 8
S = 8
T = S * B
HT = 2048
N_HT = HIDDEN // HT


def _ffn_body(x_ref, kw_ref, kb_ref, vw_ref, vb_ref, o_ref, h_ref):
    t = pl.program_id(0)
    p = pl.program_id(1)

    @pl.when(jnp.logical_and(t == 0, p == 0))
    def _init():
        o_ref[...] = jnp.broadcast_to(vb_ref[...][None], (S, T, D_MODEL))

    @pl.when(p < E)
    def _stage1():
        h = jnp.dot(x_ref[0], kw_ref[...], preferred_element_type=jnp.float32)
        h_ref[:, pl.ds(p * B, B), :] = h.reshape(S, B, HT)

    @pl.when(p >= E)
    def _stage2():
        s = p - E
        g = jax.nn.gelu(h_ref[pl.ds(s, 1), :, :][0] + kb_ref[0])
        o_ref[pl.ds(s, 1), :, :] += jnp.dot(
            g, vw_ref[...], preferred_element_type=jnp.float32
        )[None]


def kernel(x, keys_w, key_bias, values_w, value_bias):
    xe = jnp.transpose(x, (2, 1, 0, 3)).reshape(E, T, D_MODEL)
    kw2 = keys_w.reshape(D_MODEL, E * HIDDEN)
    vw2 = values_w.reshape(HIDDEN, E * D_MODEL)
    kb3 = key_bias.reshape(S, 1, HIDDEN)
    vbt = jnp.tile(value_bias, (E, 1))

    out = pl.pallas_call(
        _ffn_body,
        grid=(N_HT, E + S),
        in_specs=[
            pl.BlockSpec((1, T, D_MODEL),
                         lambda t, p: (jnp.minimum(p, E - 1), 0, 0)),
            pl.BlockSpec((D_MODEL, HT),
                         lambda t, p: (0, jnp.minimum(p, E - 1) * N_HT + t)),
            pl.BlockSpec((1, 1, HT),
                         lambda t, p: (jnp.maximum(p - E, 0), 0, t)),
            pl.BlockSpec((HT, D_MODEL),
                         lambda t, p: (t, jnp.maximum(p - E, 0))),
            pl.BlockSpec((T, D_MODEL), lambda t, p: (0, 0)),
        ],
        out_specs=pl.BlockSpec((S, T, D_MODEL), lambda t, p: (0, 0, 0)),
        out_shape=jax.ShapeDtypeStruct((S, T, D_MODEL), jnp.float32),
        scratch_shapes=[pltpu.VMEM((S, T, HT), jnp.float32)],
    )(xe, kw2, kb3, vw2, vbt)

    return out.reshape(S, E, B, D_MODEL)


# speedup vs baseline: 1.0031x; 1.0031x over previous
import jax
import jax.numpy as jnp
from jax.experimental import pallas as pl
from jax.experimental.pallas import tpu as pltpu

D_MODEL = 1024
HIDDEN = 4096
E = 8
B = 8
S = 8
T = S * B
HT = 2048
N_HT = HIDDEN // HT


def _ffn_body(x_ref, kw_ref, kb_ref, vw_ref, vb_ref, o_ref, h_ref):
    t = pl.program_id(0)
    i = pl.program_id(1)

    @pl.when(jnp.logical_and(t == 0, i == 0))
    def _init():
        o_ref[...] = jnp.broadcast_to(vb_ref[...][None], (S, T, D_MODEL))

    @pl.when(t < N_HT)
    def _stage1():
        h = jnp.dot(x_ref[0], kw_ref[...], preferred_element_type=jnp.float32)
        buf = jax.lax.rem(t, 2)
        h_ref[buf, :, pl.ds(i * B, B), :] = h.reshape(S, B, HT)

    @pl.when(t > 0)
    def _stage2():
        buf = jax.lax.rem(t + 1, 2)
        g = jax.nn.gelu(h_ref[buf, pl.ds(i, 1), :, :][0] + kb_ref[0, 0])
        o_ref[pl.ds(i, 1), :, :] += jnp.dot(
            g, vw_ref[...], preferred_element_type=jnp.float32
        )[None]


def kernel(x, keys_w, key_bias, values_w, value_bias):
    xe = jnp.transpose(x, (2, 1, 0, 3)).reshape(E, T, D_MODEL)
    kw2 = keys_w.reshape(D_MODEL, E * HIDDEN)
    vw2 = values_w.reshape(HIDDEN, S * D_MODEL)
    kb3 = key_bias.reshape(S, 1, HIDDEN)
    vbt = jnp.tile(value_bias, (E, 1))

    def kw_idx(t, i):
        return (0, jnp.where(t < N_HT,
                             i * N_HT + jnp.minimum(t, N_HT - 1),
                             (E - 1) * N_HT + N_HT - 1))

    def vw_idx(t, i):
        return (jnp.maximum(t, 1) - 1, jnp.where(t == 0, 0, i))

    out = pl.pallas_call(
        _ffn_body,
        grid=(N_HT + 1, E),
        in_specs=[
            pl.BlockSpec((1, T, D_MODEL), lambda t, i: (i, 0, 0)),
            pl.BlockSpec((D_MODEL, HT), kw_idx),
            pl.BlockSpec((1, 1, HT),
                         lambda t, i: (i, 0, jnp.maximum(t, 1) - 1)),
            pl.BlockSpec((HT, D_MODEL), vw_idx),
            pl.BlockSpec((T, D_MODEL), lambda t, i: (0, 0)),
        ],
        out_specs=pl.BlockSpec((S, T, D_MODEL), lambda t, i: (0, 0, 0)),
        out_shape=jax.ShapeDtypeStruct((S, T, D_MODEL), jnp.float32),
        scratch_shapes=[pltpu.VMEM((2, S, T, HT), jnp.float32)],
    )(xe, kw2, kb3, vw2, vbt)

    return out.reshape(S, E, B, D_MODEL)


# speedup vs baseline: 2.9591x; 2.9499x over previous
import jax
import jax.numpy as jnp
from jax.experimental import pallas as pl
from jax.experimental.pallas import tpu as pltpu

D_MODEL = 1024
HIDDEN = 4096
E = 8
B = 8
S = 8
T = S * B
HT = 256
N_HT = HIDDEN // HT


def _ffn_body(x_ref, kw_ref, kb_ref, vw_ref, vb_ref, o_ref, h_ref):
    t = pl.program_id(0)

    @pl.when(t == 0)
    def _init():
        o_ref[...] = jnp.broadcast_to(vb_ref[...][None], (S, T, D_MODEL))

    kwt = jnp.swapaxes(kw_ref[...].reshape(D_MODEL, E, HT), 0, 1)
    for e in range(E):
        h = jnp.dot(x_ref[pl.ds(e * T, T), :], kwt[e],
                    preferred_element_type=jnp.float32)
        h_ref[:, pl.ds(e * B, B), :] = h.reshape(S, B, HT)

    vwt = jnp.swapaxes(vw_ref[...].reshape(HT, S, D_MODEL), 0, 1)
    for s in range(S):
        g = jax.nn.gelu(h_ref[s] + kb_ref[s][None])
        o_ref[pl.ds(s, 1), :, :] += jnp.dot(
            g, vwt[s], preferred_element_type=jnp.float32
        )[None]


def kernel(x, keys_w, key_bias, values_w, value_bias):
    xe = jnp.transpose(x, (2, 1, 0, 3)).reshape(E * T, D_MODEL)
    kw2 = keys_w.reshape(D_MODEL * E, HIDDEN)
    vw2 = values_w.reshape(HIDDEN * S, D_MODEL)
    vbt = jnp.tile(value_bias, (E, 1))

    out = pl.pallas_call(
        _ffn_body,
        grid=(N_HT,),
        in_specs=[
            pl.BlockSpec((E * T, D_MODEL), lambda t: (0, 0)),
            pl.BlockSpec((D_MODEL * E, HT), lambda t: (0, t)),
            pl.BlockSpec((S, HT), lambda t: (0, t)),
            pl.BlockSpec((HT * S, D_MODEL), lambda t: (t, 0)),
            pl.BlockSpec((T, D_MODEL), lambda t: (0, 0)),
        ],
        out_specs=pl.BlockSpec((S, T, D_MODEL), lambda t: (0, 0, 0)),
        out_shape=jax.ShapeDtypeStruct((S, T, D_MODEL), jnp.float32),
        scratch_shapes=[pltpu.VMEM((S, T, HT), jnp.float32)],
    )(xe, kw2, key_bias, vw2, vbt)

    return out.reshape(S, E, B, D_MODEL)
